# Initial kernel scaffold; baseline (speedup 1.0000x reference)
import jax
import jax.numpy as jnp
from jax.experimental import pallas as pl


def kernel(triple, hd, td, neg_sample_r, entity_emb, relation_emb, word_emb, gate_emb, weight_h, weight_t, weight_bias):
    raise NotImplementedError("write your pallas kernel here")


# speedup vs baseline: 6.1716x; 6.1716x over previous
import functools

import jax
import jax.numpy as jnp
from jax import lax
from jax.experimental import pallas as pl
from jax.experimental.pallas import tpu as pltpu
from jax.experimental.pallas import tpu_sc as plsc

L = 16
NC = 2
NS = 16
NW = NC * NS


def _make_sc_gather(B, D, W):
    RPW = B // NW
    SPC = 2
    IDXM = SPC * W
    CPW = RPW // SPC
    GRP = 8

    def body(head_hbm, tail_hbm, hd_hbm, td_hbm, ent_hbm, gate_hbm, word_hbm,
             hout_hbm, tout_hbm,
             hidx, tidx, widx, hrows, trows, ghrows, gtrows, acc_h, acc_t,
             buf_a, buf_b, esem, wsem_a, wsem_b):
        wid = lax.axis_index("s") * NC + lax.axis_index("c")
        base = wid * RPW

        pltpu.sync_copy(head_hbm.at[pl.ds(base, RPW)], hidx)
        pltpu.sync_copy(tail_hbm.at[pl.ds(base, RPW)], tidx)
        cp_h = pltpu.async_copy(ent_hbm.at[hidx], hrows, esem)
        cp_t = pltpu.async_copy(ent_hbm.at[tidx], trows, esem)
        cp_gh = pltpu.async_copy(gate_hbm.at[hidx], ghrows, esem)
        cp_gt = pltpu.async_copy(gate_hbm.at[tidx], gtrows, esem)

        def word_phase(src_hbm, acc):
            pltpu.sync_copy(src_hbm.at[pl.ds(wid * CPW, CPW)], widx)
            pltpu.async_copy(word_hbm.at[widx.at[0]], buf_a, wsem_a)

            def g_body(g, _):
                for b in range(GRP):
                    c = g * GRP + b
                    buf, sem = (buf_a, wsem_a) if b % 2 == 0 else (buf_b, wsem_b)
                    nbuf, nsem = (buf_b, wsem_b) if b % 2 == 0 else (buf_a, wsem_a)
                    pltpu.make_async_copy(word_hbm.at[widx.at[c]], buf, sem).wait()

                    @pl.when(c + 1 < CPW)
                    def _issue_next():
                        pltpu.async_copy(word_hbm.at[widx.at[c + 1]], nbuf, nsem)

                    for s in range(SPC):
                        def j_body(j, accs, _s=s):
                            return tuple(
                                accs[k] + buf[_s * W + j, pl.ds(k * L, L)]
                                for k in range(D // L))
                        accs = lax.fori_loop(
                            0, W, j_body,
                            tuple(jnp.zeros((L,), jnp.float32)
                                  for _ in range(D // L)))
                        for k in range(D // L):
                            acc[SPC * c + s, pl.ds(k * L, L)] = accs[k]
                return 0

            lax.fori_loop(0, CPW // GRP, g_body, 0)

        word_phase(hd_hbm, acc_h)
        word_phase(td_hbm, acc_t)

        cp_h.wait()
        cp_t.wait()
        cp_gh.wait()
        cp_gt.wait()

        def c_body(r, _):
            for k in range(D // L):
                sl = pl.ds(k * L, L)
                gh = ghrows[r, sl]
                sh = 1.0 / (1.0 + jnp.exp(-gh))
                hrows[r, sl] = sh * hrows[r, sl] + (1.0 - sh) * acc_h[r, sl]
                gt = gtrows[r, sl]
                st = 1.0 / (1.0 + jnp.exp(-gt))
                trows[r, sl] = st * trows[r, sl] + (1.0 - st) * acc_t[r, sl]
            return 0

        lax.fori_loop(0, RPW, c_body, 0)
        pltpu.sync_copy(hrows, hout_hbm.at[pl.ds(base, RPW)])
        pltpu.sync_copy(trows, tout_hbm.at[pl.ds(base, RPW)])

    return pl.kernel(
        body,
        out_type=[jax.ShapeDtypeStruct((B, D), jnp.float32),
                  jax.ShapeDtypeStruct((B, D), jnp.float32)],
        mesh=plsc.VectorSubcoreMesh(core_axis_name="c", subcore_axis_name="s"),
        compiler_params=pltpu.CompilerParams(use_tc_tiling_on_sc=False),
        scratch_types=[
            pltpu.VMEM((RPW,), jnp.int32),
            pltpu.VMEM((RPW,), jnp.int32),
            pltpu.VMEM((CPW, IDXM), jnp.int32),
            pltpu.VMEM((RPW, D), jnp.float32),
            pltpu.VMEM((RPW, D), jnp.float32),
            pltpu.VMEM((RPW, D), jnp.float32),
            pltpu.VMEM((RPW, D), jnp.float32),
            pltpu.VMEM((RPW, D), jnp.float32),
            pltpu.VMEM((RPW, D), jnp.float32),
            pltpu.VMEM((IDXM, D), jnp.float32),
            pltpu.VMEM((IDXM, D), jnp.float32),
            pltpu.SemaphoreType.DMA,
            pltpu.SemaphoreType.DMA,
            pltpu.SemaphoreType.DMA,
        ],
    )


def _dense_body(h_ref, t_ref, wh_ref, wt_ref, wb_ref, rel_ref, neg_ref, o_ref):
    hrt = jnp.tanh(h_ref[...] * wh_ref[...] + t_ref[...] * wt_ref[...]
                   + wb_ref[...])
    scores = lax.dot_general(hrt, rel_ref[...], (((1,), (1,)), ((), ())),
                             preferred_element_type=jnp.float32)
    w = jnp.abs(neg_ref[...])
    m = jnp.max(w * scores, axis=1, keepdims=True)
    e = jnp.exp(scores - m)
    s = jnp.sum(e * w, axis=1, keepdims=True)
    o_ref[...] = e / s * w


def _dense_call(h_c, t_c, wh, wt, wb, relation_emb, neg_sample_r, bb=512):
    B, D = h_c.shape
    R = relation_emb.shape[0]
    return pl.pallas_call(
        _dense_body,
        grid=(B // bb,),
        in_specs=[
            pl.BlockSpec((bb, D), lambda i: (i, 0)),
            pl.BlockSpec((bb, D), lambda i: (i, 0)),
            pl.BlockSpec((1, D), lambda i: (0, 0)),
            pl.BlockSpec((1, D), lambda i: (0, 0)),
            pl.BlockSpec((1, D), lambda i: (0, 0)),
            pl.BlockSpec((R, D), lambda i: (0, 0)),
            pl.BlockSpec((bb, R), lambda i: (i, 0)),
        ],
        out_specs=pl.BlockSpec((bb, R), lambda i: (i, 0)),
        out_shape=jax.ShapeDtypeStruct((B, R), jnp.float32),
    )(h_c, t_c, wh, wt, wb, relation_emb, neg_sample_r)


def kernel(triple, hd, td, neg_sample_r, entity_emb, relation_emb, word_emb,
           gate_emb, weight_h, weight_t, weight_bias):
    B, W = hd.shape
    D = entity_emb.shape[1]
    head = triple[:, 0]
    tail = triple[:, 1]
    hd2 = hd.reshape(B * W // (2 * W), 2 * W)
    td2 = td.reshape(B * W // (2 * W), 2 * W)
    h_c, t_c = _make_sc_gather(B, D, W)(
        head, tail, hd2, td2, entity_emb, gate_emb, word_emb)
    return _dense_call(h_c, t_c, weight_h.reshape(1, D), weight_t.reshape(1, D),
                       weight_bias.reshape(1, D), relation_emb, neg_sample_r)


# speedup vs baseline: 7.1834x; 1.1639x over previous
import jax
import jax.numpy as jnp
from jax import lax
from jax.experimental import pallas as pl
from jax.experimental.pallas import tpu as pltpu
from jax.experimental.pallas import tpu_sc as plsc

L = 16
NC = 2
NS = 16
NW = NC * NS


def _make_sc_entgate(B, D2):
    RPW = B // NW

    def body(head_hbm, tail_hbm, eg_hbm, hout_hbm, tout_hbm,
             hidx, tidx, hbuf, tbuf, sem):
        wid = lax.axis_index("s") * NC + lax.axis_index("c")
        base = wid * RPW
        pltpu.sync_copy(head_hbm.at[pl.ds(base, RPW)], hidx)
        pltpu.sync_copy(tail_hbm.at[pl.ds(base, RPW)], tidx)
        cp_h = pltpu.async_copy(eg_hbm.at[hidx], hbuf, sem)
        cp_t = pltpu.async_copy(eg_hbm.at[tidx], tbuf, sem)
        cp_h.wait()
        cp_t.wait()
        pltpu.sync_copy(hbuf, hout_hbm.at[pl.ds(base, RPW)])
        pltpu.sync_copy(tbuf, tout_hbm.at[pl.ds(base, RPW)])

    return pl.kernel(
        body,
        out_type=[jax.ShapeDtypeStruct((B, D2), jnp.float32),
                  jax.ShapeDtypeStruct((B, D2), jnp.float32)],
        mesh=plsc.VectorSubcoreMesh(core_axis_name="c", subcore_axis_name="s"),
        scratch_types=[
            pltpu.VMEM((RPW,), jnp.int32),
            pltpu.VMEM((RPW,), jnp.int32),
            pltpu.VMEM((RPW, D2), jnp.float32),
            pltpu.VMEM((RPW, D2), jnp.float32),
            pltpu.SemaphoreType.DMA,
        ],
    )


def _make_sc_wordsum(B, D, W):
    RPW = B // NW
    SPC = 2
    IDXM = SPC * W
    CPW = RPW // SPC
    GRP = 8

    def body(hd_hbm, td_hbm, word_hbm, hout_hbm, tout_hbm,
             widx, acc_h, acc_t, buf_a, buf_b, wsem_a, wsem_b):
        wid = lax.axis_index("s") * NC + lax.axis_index("c")
        base = wid * RPW

        def word_phase(src_hbm, acc):
            pltpu.sync_copy(src_hbm.at[pl.ds(wid * CPW, CPW)], widx)
            pltpu.async_copy(word_hbm.at[widx.at[0]], buf_a, wsem_a)

            def g_body(g, _):
                for b in range(GRP):
                    c = g * GRP + b
                    buf, sem = (buf_a, wsem_a) if b % 2 == 0 else (buf_b, wsem_b)
                    nbuf, nsem = (buf_b, wsem_b) if b % 2 == 0 else (buf_a, wsem_a)
                    pltpu.make_async_copy(word_hbm.at[widx.at[c]], buf, sem).wait()

                    @pl.when(c + 1 < CPW)
                    def _issue_next():
                        pltpu.async_copy(word_hbm.at[widx.at[c + 1]], nbuf, nsem)

                    for s in range(SPC):
                        def j_body(j, accs, _s=s):
                            return tuple(
                                accs[k] + buf[_s * W + j, pl.ds(k * L, L)]
                                for k in range(D // L))
                        accs = lax.fori_loop(
                            0, W, j_body,
                            tuple(jnp.zeros((L,), jnp.float32)
                                  for _ in range(D // L)))
                        for k in range(D // L):
                            acc[SPC * c + s, pl.ds(k * L, L)] = accs[k]
                return 0

            lax.fori_loop(0, CPW // GRP, g_body, 0)

        word_phase(hd_hbm, acc_h)
        word_phase(td_hbm, acc_t)
        pltpu.sync_copy(acc_h, hout_hbm.at[pl.ds(base, RPW)])
        pltpu.sync_copy(acc_t, tout_hbm.at[pl.ds(base, RPW)])

    return pl.kernel(
        body,
        out_type=[jax.ShapeDtypeStruct((B, D), jnp.float32),
                  jax.ShapeDtypeStruct((B, D), jnp.float32)],
        mesh=plsc.VectorSubcoreMesh(core_axis_name="c", subcore_axis_name="s"),
        compiler_params=pltpu.CompilerParams(use_tc_tiling_on_sc=False),
        scratch_types=[
            pltpu.VMEM((CPW, IDXM), jnp.int32),
            pltpu.VMEM((RPW, D), jnp.float32),
            pltpu.VMEM((RPW, D), jnp.float32),
            pltpu.VMEM((IDXM, D), jnp.float32),
            pltpu.VMEM((IDXM, D), jnp.float32),
            pltpu.SemaphoreType.DMA,
            pltpu.SemaphoreType.DMA,
        ],
    )


def _dense_body(heg_ref, teg_ref, hde_ref, tde_ref, wh_ref, wt_ref, wb_ref,
                rel_ref, neg_ref, o_ref):
    D = hde_ref.shape[1]
    heg = heg_ref[...]
    teg = teg_ref[...]
    gh = jax.nn.sigmoid(heg[:, D:])
    gt = jax.nn.sigmoid(teg[:, D:])
    h = gh * heg[:, :D] + (1.0 - gh) * hde_ref[...]
    t = gt * teg[:, :D] + (1.0 - gt) * tde_ref[...]
    hrt = jnp.tanh(h * wh_ref[...] + t * wt_ref[...] + wb_ref[...])
    scores = lax.dot_general(hrt, rel_ref[...], (((1,), (1,)), ((), ())),
                             preferred_element_type=jnp.float32)
    w = jnp.abs(neg_ref[...])
    m = jnp.max(w * scores, axis=1, keepdims=True)
    e = jnp.exp(scores - m)
    s = jnp.sum(e * w, axis=1, keepdims=True)
    o_ref[...] = e / s * w


def _dense_call(heg, teg, hde, tde, wh, wt, wb, relation_emb, neg_sample_r,
                bb=512):
    B, D = hde.shape
    R = relation_emb.shape[0]
    return pl.pallas_call(
        _dense_body,
        grid=(B // bb,),
        in_specs=[
            pl.BlockSpec((bb, 2 * D), lambda i: (i, 0)),
            pl.BlockSpec((bb, 2 * D), lambda i: (i, 0)),
            pl.BlockSpec((bb, D), lambda i: (i, 0)),
            pl.BlockSpec((bb, D), lambda i: (i, 0)),
            pl.BlockSpec((1, D), lambda i: (0, 0)),
            pl.BlockSpec((1, D), lambda i: (0, 0)),
            pl.BlockSpec((1, D), lambda i: (0, 0)),
            pl.BlockSpec((R, D), lambda i: (0, 0)),
            pl.BlockSpec((bb, R), lambda i: (i, 0)),
        ],
        out_specs=pl.BlockSpec((bb, R), lambda i: (i, 0)),
        out_shape=jax.ShapeDtypeStruct((B, R), jnp.float32),
    )(heg, teg, hde, tde, wh, wt, wb, relation_emb, neg_sample_r)


def kernel(triple, hd, td, neg_sample_r, entity_emb, relation_emb, word_emb,
           gate_emb, weight_h, weight_t, weight_bias):
    B, W = hd.shape
    D = entity_emb.shape[1]
    head = triple[:, 0]
    tail = triple[:, 1]
    eg = jnp.concatenate([entity_emb, gate_emb], axis=1)
    hd2 = hd.reshape(B * W // (2 * W), 2 * W)
    td2 = td.reshape(B * W // (2 * W), 2 * W)
    heg, teg = _make_sc_entgate(B, 2 * D)(head, tail, eg)
    hde, tde = _make_sc_wordsum(B, D, W)(hd2, td2, word_emb)
    return _dense_call(heg, teg, hde, tde, weight_h.reshape(1, D),
                       weight_t.reshape(1, D), weight_bias.reshape(1, D),
                       relation_emb, neg_sample_r)


# speedup vs baseline: 7.4866x; 1.0422x over previous
import jax
import jax.numpy as jnp
from jax import lax
from jax.experimental import pallas as pl
from jax.experimental.pallas import tpu as pltpu
from jax.experimental.pallas import tpu_sc as plsc

L = 16
NC = 2
NS = 16
NW = NC * NS


def _make_sc_entgate(B, D2):
    RPW = B // NW

    def body(head_hbm, tail_hbm, eg_hbm, hout_hbm, tout_hbm,
             hidx, tidx, hbuf, tbuf, sem):
        wid = lax.axis_index("s") * NC + lax.axis_index("c")
        base = wid * RPW
        pltpu.sync_copy(head_hbm.at[pl.ds(base, RPW)], hidx)
        pltpu.sync_copy(tail_hbm.at[pl.ds(base, RPW)], tidx)
        cp_h = pltpu.async_copy(eg_hbm.at[hidx], hbuf, sem)
        cp_t = pltpu.async_copy(eg_hbm.at[tidx], tbuf, sem)
        cp_h.wait()
        cp_t.wait()
        pltpu.sync_copy(hbuf, hout_hbm.at[pl.ds(base, RPW)])
        pltpu.sync_copy(tbuf, tout_hbm.at[pl.ds(base, RPW)])

    return pl.kernel(
        body,
        out_type=[jax.ShapeDtypeStruct((B, D2), jnp.float32),
                  jax.ShapeDtypeStruct((B, D2), jnp.float32)],
        mesh=plsc.VectorSubcoreMesh(core_axis_name="c", subcore_axis_name="s"),
        scratch_types=[
            pltpu.VMEM((RPW,), jnp.int32),
            pltpu.VMEM((RPW,), jnp.int32),
            pltpu.VMEM((RPW, D2), jnp.float32),
            pltpu.VMEM((RPW, D2), jnp.float32),
            pltpu.SemaphoreType.DMA,
        ],
    )


def _make_sc_wordsum(B, D, W):
    RPW = B // NW
    SPC = 2
    IDXM = SPC * W
    CPW = RPW // SPC
    NBUF = 4

    def body(hd_hbm, td_hbm, word_hbm, hout_hbm, tout_hbm,
             widx, acc_h, acc_t, *bufs_sems):
        bufs = bufs_sems[:NBUF]
        sems = bufs_sems[NBUF:]
        wid = lax.axis_index("s") * NC + lax.axis_index("c")
        base = wid * RPW

        def word_phase(src_hbm, acc):
            pltpu.sync_copy(src_hbm.at[pl.ds(wid * CPW, CPW)], widx)
            for p in range(NBUF - 1):
                pltpu.async_copy(word_hbm.at[widx.at[p]], bufs[p], sems[p])

            def g_body(g, _):
                for b in range(NBUF):
                    c = g * NBUF + b
                    buf, sem = bufs[b], sems[b]
                    nb = (b + NBUF - 1) % NBUF
                    pltpu.make_async_copy(
                        word_hbm.at[widx.at[c]], buf, sem).wait()

                    @pl.when(c + NBUF - 1 < CPW)
                    def _issue_next():
                        pltpu.async_copy(
                            word_hbm.at[widx.at[c + NBUF - 1]],
                            bufs[nb], sems[nb])

                    for s in range(SPC):
                        def j_body(j, accs, _s=s):
                            return tuple(
                                accs[k] + buf[_s * W + j, pl.ds(k * L, L)]
                                for k in range(D // L))
                        accs = lax.fori_loop(
                            0, W, j_body,
                            tuple(jnp.zeros((L,), jnp.float32)
                                  for _ in range(D // L)))
                        row = SPC * c + s
                        for k in range(D // L):
                            acc[row, pl.ds(k * L, L)] = accs[k]
                return 0

            lax.fori_loop(0, CPW // NBUF, g_body, 0)

        word_phase(hd_hbm, acc_h)
        word_phase(td_hbm, acc_t)
        pltpu.sync_copy(acc_h, hout_hbm.at[pl.ds(base, RPW)])
        pltpu.sync_copy(acc_t, tout_hbm.at[pl.ds(base, RPW)])

    return pl.kernel(
        body,
        out_type=[jax.ShapeDtypeStruct((B, D), jnp.float32),
                  jax.ShapeDtypeStruct((B, D), jnp.float32)],
        mesh=plsc.VectorSubcoreMesh(core_axis_name="c", subcore_axis_name="s"),
        compiler_params=pltpu.CompilerParams(use_tc_tiling_on_sc=False),
        scratch_types=[
            pltpu.VMEM((CPW, IDXM), jnp.int32),
            pltpu.VMEM((RPW, D), jnp.float32),
            pltpu.VMEM((RPW, D), jnp.float32),
        ] + [pltpu.VMEM((IDXM, D), jnp.float32) for _ in range(NBUF)]
          + [pltpu.SemaphoreType.DMA for _ in range(NBUF)],
    )


def _concat_body(a_ref, b_ref, o_ref):
    D = a_ref.shape[1]
    o_ref[:, :D] = a_ref[...]
    o_ref[:, D:] = b_ref[...]


def _concat_call(a, b, br=2000):
    N, D = a.shape
    return pl.pallas_call(
        _concat_body,
        grid=(N // br,),
        in_specs=[
            pl.BlockSpec((br, D), lambda i: (i, 0)),
            pl.BlockSpec((br, D), lambda i: (i, 0)),
        ],
        out_specs=pl.BlockSpec((br, 2 * D), lambda i: (i, 0)),
        out_shape=jax.ShapeDtypeStruct((N, 2 * D), jnp.float32),
    )(a, b)


def _dense_body(heg_ref, teg_ref, hde_ref, tde_ref, wh_ref, wt_ref, wb_ref,
                rel_ref, neg_ref, o_ref):
    D = hde_ref.shape[1]
    heg = heg_ref[...]
    teg = teg_ref[...]
    gh = jax.nn.sigmoid(heg[:, D:])
    gt = jax.nn.sigmoid(teg[:, D:])
    h = gh * heg[:, :D] + (1.0 - gh) * hde_ref[...]
    t = gt * teg[:, :D] + (1.0 - gt) * tde_ref[...]
    hrt = jnp.tanh(h * wh_ref[...] + t * wt_ref[...] + wb_ref[...])
    scores = lax.dot_general(hrt, rel_ref[...], (((1,), (1,)), ((), ())),
                             preferred_element_type=jnp.float32)
    w = jnp.abs(neg_ref[...])
    m = jnp.max(w * scores, axis=1, keepdims=True)
    e = jnp.exp(scores - m)
    s = jnp.sum(e * w, axis=1, keepdims=True)
    o_ref[...] = e / s * w


def _dense_call(heg, teg, hde, tde, wh, wt, wb, relation_emb, neg_sample_r,
                bb=512):
    B, D = hde.shape
    R = relation_emb.shape[0]
    return pl.pallas_call(
        _dense_body,
        grid=(B // bb,),
        in_specs=[
            pl.BlockSpec((bb, 2 * D), lambda i: (i, 0)),
            pl.BlockSpec((bb, 2 * D), lambda i: (i, 0)),
            pl.BlockSpec((bb, D), lambda i: (i, 0)),
            pl.BlockSpec((bb, D), lambda i: (i, 0)),
            pl.BlockSpec((1, D), lambda i: (0, 0)),
            pl.BlockSpec((1, D), lambda i: (0, 0)),
            pl.BlockSpec((1, D), lambda i: (0, 0)),
            pl.BlockSpec((R, D), lambda i: (0, 0)),
            pl.BlockSpec((bb, R), lambda i: (i, 0)),
        ],
        out_specs=pl.BlockSpec((bb, R), lambda i: (i, 0)),
        out_shape=jax.ShapeDtypeStruct((B, R), jnp.float32),
    )(heg, teg, hde, tde, wh, wt, wb, relation_emb, neg_sample_r)


def kernel(triple, hd, td, neg_sample_r, entity_emb, relation_emb, word_emb,
           gate_emb, weight_h, weight_t, weight_bias):
    B, W = hd.shape
    D = entity_emb.shape[1]
    head = triple[:, 0]
    tail = triple[:, 1]
    eg = _concat_call(entity_emb, gate_emb)
    hd2 = hd.reshape(B * W // (2 * W), 2 * W)
    td2 = td.reshape(B * W // (2 * W), 2 * W)
    heg, teg = _make_sc_entgate(B, 2 * D)(head, tail, eg)
    hde, tde = _make_sc_wordsum(B, D, W)(hd2, td2, word_emb)
    return _dense_call(heg, teg, hde, tde, weight_h.reshape(1, D),
                       weight_t.reshape(1, D), weight_bias.reshape(1, D),
                       relation_emb, neg_sample_r)


# speedup vs baseline: 8.0763x; 1.0788x over previous
import jax
import jax.numpy as jnp
from jax import lax
from jax.experimental import pallas as pl
from jax.experimental.pallas import tpu as pltpu
from jax.experimental.pallas import tpu_sc as plsc

L = 16
NC = 2
NS = 16
NW = NC * NS


def _make_sc_gather(B, D, W):
    RPW = B // NW
    SPC = 2
    IDXM = SPC * W
    CPW = RPW // SPC
    NBUF = 4

    def body(head_hbm, tail_hbm, hd_hbm, td_hbm, eg_hbm, word_hbm,
             heg_hbm, teg_hbm, wsum_hbm,
             hidx, tidx, widx, hrows, trows, acc, *bufs_sems):
        bufs = bufs_sems[:NBUF]
        sems = bufs_sems[NBUF:NBUF + NBUF]
        esem = bufs_sems[-1]
        wid = lax.axis_index("s") * NC + lax.axis_index("c")
        base = wid * RPW

        pltpu.sync_copy(head_hbm.at[pl.ds(base, RPW)], hidx)
        pltpu.sync_copy(tail_hbm.at[pl.ds(base, RPW)], tidx)
        cp_h = pltpu.async_copy(eg_hbm.at[hidx], hrows, esem)
        cp_t = pltpu.async_copy(eg_hbm.at[tidx], trows, esem)

        def word_phase(src_hbm, col0):
            pltpu.sync_copy(src_hbm.at[pl.ds(wid * CPW, CPW)], widx)
            for p in range(NBUF - 1):
                pltpu.async_copy(word_hbm.at[widx.at[p]], bufs[p], sems[p])

            def g_body(g, _):
                for b in range(NBUF):
                    c = g * NBUF + b
                    buf, sem = bufs[b], sems[b]
                    nb = (b + NBUF - 1) % NBUF
                    pltpu.make_async_copy(
                        word_hbm.at[widx.at[c]], buf, sem).wait()

                    @pl.when(c + NBUF - 1 < CPW)
                    def _issue_next():
                        pltpu.async_copy(
                            word_hbm.at[widx.at[c + NBUF - 1]],
                            bufs[nb], sems[nb])

                    for s in range(SPC):
                        def j_body(j, accs, _s=s):
                            return tuple(
                                accs[k] + buf[_s * W + j, pl.ds(k * L, L)]
                                for k in range(D // L))
                        accs = lax.fori_loop(
                            0, W, j_body,
                            tuple(jnp.zeros((L,), jnp.float32)
                                  for _ in range(D // L)))
                        row = SPC * c + s
                        for k in range(D // L):
                            acc[row, pl.ds(col0 + k * L, L)] = accs[k]
                return 0

            lax.fori_loop(0, CPW // NBUF, g_body, 0)

        word_phase(hd_hbm, 0)
        word_phase(td_hbm, D)

        cp_h.wait()
        cp_t.wait()
        pltpu.sync_copy(hrows, heg_hbm.at[pl.ds(base, RPW)])
        pltpu.sync_copy(trows, teg_hbm.at[pl.ds(base, RPW)])
        pltpu.sync_copy(acc, wsum_hbm.at[pl.ds(base, RPW)])

    return pl.kernel(
        body,
        out_type=[jax.ShapeDtypeStruct((B, 2 * D), jnp.float32),
                  jax.ShapeDtypeStruct((B, 2 * D), jnp.float32),
                  jax.ShapeDtypeStruct((B, 2 * D), jnp.float32)],
        mesh=plsc.VectorSubcoreMesh(core_axis_name="c", subcore_axis_name="s"),
        compiler_params=pltpu.CompilerParams(use_tc_tiling_on_sc=False),
        scratch_types=[
            pltpu.VMEM((RPW,), jnp.int32),
            pltpu.VMEM((RPW,), jnp.int32),
            pltpu.VMEM((CPW, IDXM), jnp.int32),
            pltpu.VMEM((RPW, 2 * D), jnp.float32),
            pltpu.VMEM((RPW, 2 * D), jnp.float32),
            pltpu.VMEM((RPW, 2 * D), jnp.float32),
        ] + [pltpu.VMEM((IDXM, D), jnp.float32) for _ in range(NBUF)]
          + [pltpu.SemaphoreType.DMA for _ in range(NBUF)]
          + [pltpu.SemaphoreType.DMA],
    )


def _dense_body(heg_ref, teg_ref, ws_ref, wh_ref, wt_ref, wb_ref,
                rel_ref, neg_ref, o_ref):
    D = wh_ref.shape[1]
    heg = heg_ref[...]
    teg = teg_ref[...]
    ws = ws_ref[...]
    gh = jax.nn.sigmoid(heg[:, D:])
    gt = jax.nn.sigmoid(teg[:, D:])
    h = gh * heg[:, :D] + (1.0 - gh) * ws[:, :D]
    t = gt * teg[:, :D] + (1.0 - gt) * ws[:, D:]
    hrt = jnp.tanh(h * wh_ref[...] + t * wt_ref[...] + wb_ref[...])
    scores = lax.dot_general(hrt, rel_ref[...], (((1,), (1,)), ((), ())),
                             preferred_element_type=jnp.float32)
    w = jnp.abs(neg_ref[...])
    m = jnp.max(w * scores, axis=1, keepdims=True)
    e = jnp.exp(scores - m)
    s = jnp.sum(e * w, axis=1, keepdims=True)
    o_ref[...] = e / s * w


def _dense_call(heg, teg, wsum, wh, wt, wb, relation_emb, neg_sample_r,
                bb=512):
    B = heg.shape[0]
    D = wh.shape[1]
    R = relation_emb.shape[0]
    return pl.pallas_call(
        _dense_body,
        grid=(B // bb,),
        in_specs=[
            pl.BlockSpec((bb, 2 * D), lambda i: (i, 0)),
            pl.BlockSpec((bb, 2 * D), lambda i: (i, 0)),
            pl.BlockSpec((bb, 2 * D), lambda i: (i, 0)),
            pl.BlockSpec((1, D), lambda i: (0, 0)),
            pl.BlockSpec((1, D), lambda i: (0, 0)),
            pl.BlockSpec((1, D), lambda i: (0, 0)),
            pl.BlockSpec((R, D), lambda i: (0, 0)),
            pl.BlockSpec((bb, R), lambda i: (i, 0)),
        ],
        out_specs=pl.BlockSpec((bb, R), lambda i: (i, 0)),
        out_shape=jax.ShapeDtypeStruct((B, R), jnp.float32),
    )(heg, teg, wsum, wh, wt, wb, relation_emb, neg_sample_r)


def kernel(triple, hd, td, neg_sample_r, entity_emb, relation_emb, word_emb,
           gate_emb, weight_h, weight_t, weight_bias):
    B, W = hd.shape
    D = entity_emb.shape[1]
    head = triple[:, 0]
    tail = triple[:, 1]
    eg = jnp.concatenate([entity_emb, gate_emb], axis=1)
    hd2 = hd.reshape(B * W // (2 * W), 2 * W)
    td2 = td.reshape(B * W // (2 * W), 2 * W)
    heg, teg, wsum = _make_sc_gather(B, D, W)(
        head, tail, hd2, td2, eg, word_emb)
    return _dense_call(heg, teg, wsum, weight_h.reshape(1, D),
                       weight_t.reshape(1, D), weight_bias.reshape(1, D),
                       relation_emb, neg_sample_r)


# speedup vs baseline: 11.1798x; 1.3843x over previous
import jax
import jax.numpy as jnp
from jax import lax
from jax.experimental import pallas as pl
from jax.experimental.pallas import tpu as pltpu
from jax.experimental.pallas import tpu_sc as plsc

L = 16
NC = 2
NS = 16
NW = NC * NS


def _tables_body(entt_ref, gatet_ref, wordt_ref, eg_ref, z_ref):
    et = entt_ref[...].T
    gt = gatet_ref[...].T
    D = et.shape[1]
    eg_ref[:, :D] = et
    eg_ref[:, D:] = gt
    wt = wordt_ref[...].T
    z_ref[:, :D] = wt
    z_ref[:, D:] = wt


def _tables_call(entt, gatet, wordt, bn=2048):
    D, N = entt.shape
    return pl.pallas_call(
        _tables_body,
        grid=((N + bn - 1) // bn,),
        in_specs=[
            pl.BlockSpec((D, bn), lambda i: (0, i)),
            pl.BlockSpec((D, bn), lambda i: (0, i)),
            pl.BlockSpec((D, bn), lambda i: (0, i)),
        ],
        out_specs=[
            pl.BlockSpec((bn, 2 * D), lambda i: (i, 0)),
            pl.BlockSpec((bn, 2 * D), lambda i: (i, 0)),
        ],
        out_shape=[jax.ShapeDtypeStruct((N, 2 * D), jnp.float32),
                   jax.ShapeDtypeStruct((N, 2 * D), jnp.float32)],
    )(entt, gatet, wordt)


def _make_sc_gather(B, D, W):
    RPW = B // NW
    SPC = 2
    IDXM = SPC * W
    CPW = RPW // SPC
    NBUF = 4

    def body(head_hbm, tail_hbm, hd_hbm, td_hbm, eg_hbm, word_hbm,
             heg_hbm, teg_hbm, wsum_hbm,
             hidx, tidx, widx, hrows, trows, acc, *bufs_sems):
        bufs = bufs_sems[:NBUF]
        sems = bufs_sems[NBUF:NBUF + NBUF]
        esem = bufs_sems[-1]
        wid = lax.axis_index("s") * NC + lax.axis_index("c")
        base = wid * RPW

        pltpu.sync_copy(head_hbm.at[pl.ds(base, RPW)], hidx)
        pltpu.sync_copy(tail_hbm.at[pl.ds(base, RPW)], tidx)
        cp_h = pltpu.async_copy(eg_hbm.at[hidx], hrows, esem)
        cp_t = pltpu.async_copy(eg_hbm.at[tidx], trows, esem)

        def word_phase(src_hbm, col0):
            pltpu.sync_copy(src_hbm.at[pl.ds(wid * CPW, CPW)], widx)
            for p in range(NBUF - 1):
                pltpu.async_copy(word_hbm.at[widx.at[p]], bufs[p], sems[p])

            def g_body(g, _):
                for b in range(NBUF):
                    c = g * NBUF + b
                    buf, sem = bufs[b], sems[b]
                    nb = (b + NBUF - 1) % NBUF
                    pltpu.make_async_copy(
                        word_hbm.at[widx.at[c]], buf, sem).wait()

                    @pl.when(c + NBUF - 1 < CPW)
                    def _issue_next():
                        pltpu.async_copy(
                            word_hbm.at[widx.at[c + NBUF - 1]],
                            bufs[nb], sems[nb])

                    for s in range(SPC):
                        def j_body(j, accs, _s=s):
                            return tuple(
                                accs[k] + buf[_s * W + j, pl.ds(k * L, L)]
                                for k in range(D // L))
                        accs = lax.fori_loop(
                            0, W, j_body,
                            tuple(jnp.zeros((L,), jnp.float32)
                                  for _ in range(D // L)))
                        row = SPC * c + s
                        for k in range(D // L):
                            acc[row, pl.ds(col0 + k * L, L)] = accs[k]
                return 0

            lax.fori_loop(0, CPW // NBUF, g_body, 0)

        word_phase(hd_hbm, 0)
        word_phase(td_hbm, D)

        cp_h.wait()
        cp_t.wait()
        pltpu.sync_copy(hrows, heg_hbm.at[pl.ds(base, RPW)])
        pltpu.sync_copy(trows, teg_hbm.at[pl.ds(base, RPW)])
        pltpu.sync_copy(acc, wsum_hbm.at[pl.ds(base, RPW)])

    return pl.kernel(
        body,
        out_type=[jax.ShapeDtypeStruct((B, 2 * D), jnp.float32),
                  jax.ShapeDtypeStruct((B, 2 * D), jnp.float32),
                  jax.ShapeDtypeStruct((B, 2 * D), jnp.float32)],
        mesh=plsc.VectorSubcoreMesh(core_axis_name="c", subcore_axis_name="s"),
        compiler_params=pltpu.CompilerParams(use_tc_tiling_on_sc=False),
        scratch_types=[
            pltpu.VMEM((RPW,), jnp.int32),
            pltpu.VMEM((RPW,), jnp.int32),
            pltpu.VMEM((CPW, IDXM), jnp.int32),
            pltpu.VMEM((RPW, 2 * D), jnp.float32),
            pltpu.VMEM((RPW, 2 * D), jnp.float32),
            pltpu.VMEM((RPW, 2 * D), jnp.float32),
        ] + [pltpu.VMEM((IDXM, D), jnp.float32) for _ in range(NBUF)]
          + [pltpu.SemaphoreType.DMA for _ in range(NBUF)]
          + [pltpu.SemaphoreType.DMA],
    )


def _dense_body(heg_ref, teg_ref, ws_ref, wh_ref, wt_ref, wb_ref,
                relt_ref, negt_ref, ot_ref):
    D = wh_ref.shape[1]
    heg = heg_ref[...]
    teg = teg_ref[...]
    ws = ws_ref[...]
    gh = jax.nn.sigmoid(heg[:, D:])
    gt = jax.nn.sigmoid(teg[:, D:])
    h = gh * heg[:, :D] + (1.0 - gh) * ws[:, :D]
    t = gt * teg[:, :D] + (1.0 - gt) * ws[:, D:]
    hrt = jnp.tanh(h * wh_ref[...] + t * wt_ref[...] + wb_ref[...])
    scores = lax.dot_general(relt_ref[...], hrt, (((0,), (1,)), ((), ())),
                             preferred_element_type=jnp.float32)
    w = jnp.abs(negt_ref[...])
    m = jnp.max(w * scores, axis=0, keepdims=True)
    e = jnp.exp(scores - m)
    s = jnp.sum(e * w, axis=0, keepdims=True)
    ot_ref[...] = e / s * w


def _dense_call(heg, teg, wsum, wh, wt, wb, relt, negt, bb=512):
    B = heg.shape[0]
    D = wh.shape[1]
    R = negt.shape[0]
    return pl.pallas_call(
        _dense_body,
        grid=(B // bb,),
        in_specs=[
            pl.BlockSpec((bb, 2 * D), lambda i: (i, 0)),
            pl.BlockSpec((bb, 2 * D), lambda i: (i, 0)),
            pl.BlockSpec((bb, 2 * D), lambda i: (i, 0)),
            pl.BlockSpec((1, D), lambda i: (0, 0)),
            pl.BlockSpec((1, D), lambda i: (0, 0)),
            pl.BlockSpec((1, D), lambda i: (0, 0)),
            pl.BlockSpec((D, R), lambda i: (0, 0)),
            pl.BlockSpec((R, bb), lambda i: (0, i)),
        ],
        out_specs=pl.BlockSpec((R, bb), lambda i: (0, i)),
        out_shape=jax.ShapeDtypeStruct((R, B), jnp.float32),
    )(heg, teg, wsum, wh, wt, wb, relt, negt)


def kernel(triple, hd, td, neg_sample_r, entity_emb, relation_emb, word_emb,
           gate_emb, weight_h, weight_t, weight_bias):
    B, W = hd.shape
    D = entity_emb.shape[1]
    V = entity_emb.shape[0]
    head = triple[:, 0]
    tail = triple[:, 1]
    eg, z = _tables_call(entity_emb.T, gate_emb.T, word_emb.T)
    word_lin = z.reshape(2 * V, D)
    hd2 = (hd * 2).reshape(B * W // (2 * W), 2 * W)
    td2 = (td * 2).reshape(B * W // (2 * W), 2 * W)
    heg, teg, wsum = _make_sc_gather(B, D, W)(
        head, tail, hd2, td2, eg, word_lin)
    out_t = _dense_call(heg, teg, wsum, weight_h.reshape(1, D),
                        weight_t.reshape(1, D), weight_bias.reshape(1, D),
                        relation_emb.T, neg_sample_r.T)
    return out_t.T
